# baseline (speedup 1.0000x reference)
import functools

import jax
import jax.numpy as jnp
from jax import lax
from jax.experimental import pallas as pl
from jax.experimental.pallas import tpu as pltpu
from jax.experimental.pallas import tpu_sc as plsc

_B = 128
_V = 100000
_NW = 32
_RPT = _B // _NW
_C = 2000
_NC = _V // _C
_VPC = _C // 16


def _body(x_hbm, tgt_hbm, out_ref, tgt_v, buf_v, tvchunk_v, acc_v, part_v,
          red_v, tot_v, shared):
    cid = lax.axis_index("c")
    sid = lax.axis_index("s")
    wid = cid * 16 + sid

    pltpu.sync_copy(tgt_hbm, tgt_v)
    iota = lax.iota(jnp.int32, 16)

    zero16f = jnp.zeros((16,), jnp.float32)
    top1 = jnp.float32(0.0)
    top5 = jnp.float32(0.0)
    for j in range(_RPT):
        r = wid * _RPT + j
        t = tgt_v[pl.ds(r, 1)][0]
        t_base = jnp.minimum((t // 8) * 8, _V - 16)
        flat_tb = pl.multiple_of(r * _V + t_base, 8)
        pltpu.sync_copy(x_hbm.at[pl.ds(flat_tb, 16)], tvchunk_v)
        tv = tvchunk_v[pl.ds(t - t_base, 1)][0]

        def chunk_body(c, acc):
            base = pl.multiple_of(r * _V + c * _C, 8)
            pltpu.sync_copy(x_hbm.at[pl.ds(base, _C)], buf_v)

            def vec_body(k2, acc2):
                v = buf_v[pl.ds(k2 * 16, 16)]
                col = iota + (c * _C + k2 * 16)
                m = (v > tv) | ((v == tv) & (col < t))
                return acc2 + jnp.where(m, 1, 0)

            return lax.fori_loop(0, _VPC, vec_body, acc)

        acc = lax.fori_loop(0, _NC, chunk_body, jnp.zeros((16,), jnp.int32))
        acc_v[...] = acc
        rank = acc_v[pl.ds(0, 1)][0]
        for i in range(1, 16):
            rank = rank + acc_v[pl.ds(i, 1)][0]
        top1 = top1 + jnp.where(rank < 1, 1.0, 0.0).astype(jnp.float32)
        top5 = top5 + jnp.where(rank < 5, 1.0, 0.0).astype(jnp.float32)

    part = jnp.where(iota == 0, top1,
                     jnp.where(iota == 1, top5, zero16f))
    part_v[...] = part
    pltpu.sync_copy(part_v, shared.at[sid])
    plsc.subcore_barrier()

    @pl.when(sid == 0)
    def _():
        def red(i, a):
            pltpu.sync_copy(shared.at[i], red_v)
            return a + red_v[...]

        tot = lax.fori_loop(0, 16, red, jnp.zeros((16,), jnp.float32))
        tot_v[...] = tot
        pltpu.sync_copy(tot_v, out_ref.at[cid])


@jax.jit
def _run(outputs, targets):
    mesh = plsc.VectorSubcoreMesh(core_axis_name="c", subcore_axis_name="s")
    f = functools.partial(
        pl.kernel,
        mesh=mesh,
        out_type=jax.ShapeDtypeStruct((2, 16), jnp.float32),
        scratch_types=[
            pltpu.VMEM((_B,), jnp.int32),
            pltpu.VMEM((_C,), jnp.float32),
            pltpu.VMEM((16,), jnp.float32),
            pltpu.VMEM((16,), jnp.int32),
            pltpu.VMEM((16,), jnp.float32),
            pltpu.VMEM((16,), jnp.float32),
            pltpu.VMEM((16,), jnp.float32),
            pltpu.VMEM_SHARED((16, 16), jnp.float32),
        ],
    )(_body)
    return f(outputs, targets)


def kernel(outputs, targets):
    out = _run(outputs.reshape(-1), targets)
    s = out[0] + out[1]
    return (s[0], s[1])


# speedup vs baseline: 1.6310x; 1.6310x over previous
import functools

import jax
import jax.numpy as jnp
from jax import lax
from jax.experimental import pallas as pl
from jax.experimental.pallas import tpu as pltpu
from jax.experimental.pallas import tpu_sc as plsc

_B = 128
_V = 100000
_NW = 32
_RPT = _B // _NW
_C = 2000
_NCH = _V // _C
_VPC = _C // 16


def _body(x_hbm, tgt_hbm, out_ref, tgt_v, buf0_v, buf1_v, tvchunk_v, acc_v,
          part_v, red_v, tot_v, shared, sem0, sem1):
    cid = lax.axis_index("c")
    sid = lax.axis_index("s")
    wid = cid * 16 + sid

    pltpu.sync_copy(tgt_hbm, tgt_v)
    iota = lax.iota(jnp.int32, 16)
    zero16i = jnp.zeros((16,), jnp.int32)
    one16i = jnp.ones((16,), jnp.int32)

    def row_body(r, carry):
        top1, top5 = carry
        t = tgt_v[pl.ds(r, 1)][0]
        row_off = r * _V
        t_base = jnp.minimum((t // 8) * 8, _V - 16)
        pltpu.sync_copy(
            x_hbm.at[pl.ds(pl.multiple_of(row_off + t_base, 8), 16)],
            tvchunk_v)
        tv = tvchunk_v[pl.ds(t - t_base, 1)][0]
        cb = t // _C

        def start(c, buf, sem):
            base = pl.multiple_of(row_off + c * _C, 8)
            pltpu.async_copy(x_hbm.at[pl.ds(base, _C)], buf, sem)

        def wait(buf, sem):
            pltpu.make_async_copy(x_hbm.at[pl.ds(0, _C)], buf, sem).wait()

        def process(c, buf):
            @pl.when(c < cb)
            def _():
                acc = zero16i
                for k in range(_VPC):
                    v = buf[pl.ds(k * 16, 16)]
                    acc = acc + jnp.where(v >= tv, one16i, zero16i)
                acc_v[...] = acc_v[...] + acc

            @pl.when(c > cb)
            def _():
                acc = zero16i
                for k in range(_VPC):
                    v = buf[pl.ds(k * 16, 16)]
                    acc = acc + jnp.where(v > tv, one16i, zero16i)
                acc_v[...] = acc_v[...] + acc

            @pl.when(c == cb)
            def _():
                acc = zero16i
                col = iota + c * _C
                for k in range(_VPC):
                    v = buf[pl.ds(k * 16, 16)]
                    m = (v > tv) | ((v == tv) & (col < t))
                    acc = acc + jnp.where(m, one16i, zero16i)
                    col = col + 16
                acc_v[...] = acc_v[...] + acc

        acc_v[...] = zero16i
        start(0, buf0_v, sem0)
        start(1, buf1_v, sem1)

        def pair(p, _):
            c0 = 2 * p
            wait(buf0_v, sem0)
            process(c0, buf0_v)

            @pl.when(c0 + 2 < _NCH)
            def _():
                start(c0 + 2, buf0_v, sem0)

            wait(buf1_v, sem1)
            process(c0 + 1, buf1_v)

            @pl.when(c0 + 3 < _NCH)
            def _():
                start(c0 + 3, buf1_v, sem1)

            return 0

        lax.fori_loop(0, _NCH // 2, pair, 0)

        rank = acc_v[pl.ds(0, 1)][0]
        for i in range(1, 16):
            rank = rank + acc_v[pl.ds(i, 1)][0]
        top1 = top1 + jnp.where(rank < 1, 1.0, 0.0).astype(jnp.float32)
        top5 = top5 + jnp.where(rank < 5, 1.0, 0.0).astype(jnp.float32)
        return (top1, top5)

    top1, top5 = lax.fori_loop(wid * _RPT, (wid + 1) * _RPT, row_body,
                               (jnp.float32(0.0), jnp.float32(0.0)))

    zero16f = jnp.zeros((16,), jnp.float32)
    part = jnp.where(iota == 0, top1,
                     jnp.where(iota == 1, top5, zero16f))
    part_v[...] = part
    pltpu.sync_copy(part_v, shared.at[sid])
    plsc.subcore_barrier()

    @pl.when(sid == 0)
    def _():
        def red(i, a):
            pltpu.sync_copy(shared.at[i], red_v)
            return a + red_v[...]

        tot = lax.fori_loop(0, 16, red, jnp.zeros((16,), jnp.float32))
        tot_v[...] = tot
        pltpu.sync_copy(tot_v, out_ref.at[cid])


@jax.jit
def _run(outputs, targets):
    mesh = plsc.VectorSubcoreMesh(core_axis_name="c", subcore_axis_name="s")
    f = functools.partial(
        pl.kernel,
        mesh=mesh,
        out_type=jax.ShapeDtypeStruct((2, 16), jnp.float32),
        scratch_types=[
            pltpu.VMEM((_B,), jnp.int32),
            pltpu.VMEM((_C,), jnp.float32),
            pltpu.VMEM((_C,), jnp.float32),
            pltpu.VMEM((16,), jnp.float32),
            pltpu.VMEM((16,), jnp.int32),
            pltpu.VMEM((16,), jnp.float32),
            pltpu.VMEM((16,), jnp.float32),
            pltpu.VMEM((16,), jnp.float32),
            pltpu.VMEM_SHARED((16, 16), jnp.float32),
            pltpu.SemaphoreType.DMA,
            pltpu.SemaphoreType.DMA,
        ],
    )(_body)
    return f(outputs, targets)


def kernel(outputs, targets):
    out = _run(outputs.reshape(-1), targets)
    s = out[0] + out[1]
    return (s[0], s[1])


# speedup vs baseline: 1.7258x; 1.0581x over previous
import functools

import jax
import jax.numpy as jnp
from jax import lax
from jax.experimental import pallas as pl
from jax.experimental.pallas import tpu as pltpu
from jax.experimental.pallas import tpu_sc as plsc

_B = 128
_V = 100000
_NW = 32
_RPT = _B // _NW
_C = 10000
_NCH = _V // _C
_G = 125
_NG = _C // (16 * _G)


def _body(x_hbm, tgt_hbm, out_ref, tgt_v, buf0_v, buf1_v, tvchunk_v, acc_v,
          part_v, red_v, tot_v, shared, sem0, sem1):
    cid = lax.axis_index("c")
    sid = lax.axis_index("s")
    wid = cid * 16 + sid

    pltpu.sync_copy(tgt_hbm, tgt_v)
    iota = lax.iota(jnp.int32, 16)
    zero16i = jnp.zeros((16,), jnp.int32)
    one16i = jnp.ones((16,), jnp.int32)

    def row_body(r, carry):
        top1, top5 = carry
        t = tgt_v[pl.ds(r, 1)][0]
        t_base = jnp.minimum((t // 8) * 8, _V - 16)
        pltpu.sync_copy(x_hbm.at[r, pl.ds(pl.multiple_of(t_base, 8), 16)],
                        tvchunk_v)
        tv = tvchunk_v[pl.ds(t - t_base, 1)][0]
        cb = t // _C

        def start(c, buf, sem):
            base = pl.multiple_of(c * _C, 8)
            pltpu.async_copy(x_hbm.at[r, pl.ds(base, _C)], buf, sem)

        def wait(buf, sem):
            pltpu.make_async_copy(x_hbm.at[0, pl.ds(0, _C)], buf, sem).wait()

        def count_grp(buf, gbase, cmp):
            a = [zero16i, zero16i, zero16i, zero16i]
            for k in range(_G):
                v = buf[pl.ds(gbase + k * 16, 16)]
                a[k % 4] = a[k % 4] + jnp.where(cmp(v), one16i, zero16i)
            return (a[0] + a[1]) + (a[2] + a[3])

        def process(c, buf):
            @pl.when(c < cb)
            def _():
                def grp(g, acc):
                    return acc + count_grp(buf, g * (16 * _G),
                                           lambda v: v >= tv)
                acc = lax.fori_loop(0, _NG, grp, zero16i)
                acc_v[...] = acc_v[...] + acc

            @pl.when(c > cb)
            def _():
                def grp(g, acc):
                    return acc + count_grp(buf, g * (16 * _G),
                                           lambda v: v > tv)
                acc = lax.fori_loop(0, _NG, grp, zero16i)
                acc_v[...] = acc_v[...] + acc

            @pl.when(c == cb)
            def _():
                def grp(g, acc):
                    gbase = g * (16 * _G)
                    col0 = iota + (c * _C + gbase)
                    a = [zero16i, zero16i, zero16i, zero16i]
                    col = col0
                    for k in range(_G):
                        v = buf[pl.ds(gbase + k * 16, 16)]
                        m = (v > tv) | ((v == tv) & (col < t))
                        a[k % 4] = a[k % 4] + jnp.where(m, one16i, zero16i)
                        col = col + 16
                    return acc + ((a[0] + a[1]) + (a[2] + a[3]))

                acc = lax.fori_loop(0, _NG, grp, zero16i)
                acc_v[...] = acc_v[...] + acc

        acc_v[...] = zero16i
        start(0, buf0_v, sem0)
        start(1, buf1_v, sem1)

        def pair(p, _):
            c0 = 2 * p
            wait(buf0_v, sem0)
            process(c0, buf0_v)

            @pl.when(c0 + 2 < _NCH)
            def _():
                start(c0 + 2, buf0_v, sem0)

            wait(buf1_v, sem1)
            process(c0 + 1, buf1_v)

            @pl.when(c0 + 3 < _NCH)
            def _():
                start(c0 + 3, buf1_v, sem1)

            return 0

        lax.fori_loop(0, _NCH // 2, pair, 0)

        rank = acc_v[pl.ds(0, 1)][0]
        for i in range(1, 16):
            rank = rank + acc_v[pl.ds(i, 1)][0]
        top1 = top1 + jnp.where(rank < 1, 1.0, 0.0).astype(jnp.float32)
        top5 = top5 + jnp.where(rank < 5, 1.0, 0.0).astype(jnp.float32)
        return (top1, top5)

    top1, top5 = lax.fori_loop(wid * _RPT, (wid + 1) * _RPT, row_body,
                               (jnp.float32(0.0), jnp.float32(0.0)))

    zero16f = jnp.zeros((16,), jnp.float32)
    part = jnp.where(iota == 0, top1,
                     jnp.where(iota == 1, top5, zero16f))
    part_v[...] = part
    pltpu.sync_copy(part_v, shared.at[sid])
    plsc.subcore_barrier()

    @pl.when(sid == 0)
    def _():
        def red(i, a):
            pltpu.sync_copy(shared.at[i], red_v)
            return a + red_v[...]

        tot = lax.fori_loop(0, 16, red, jnp.zeros((16,), jnp.float32))
        tot_v[...] = tot
        pltpu.sync_copy(tot_v, out_ref.at[cid])


@jax.jit
def _run(outputs, targets):
    mesh = plsc.VectorSubcoreMesh(core_axis_name="c", subcore_axis_name="s")
    f = functools.partial(
        pl.kernel,
        mesh=mesh,
        out_type=jax.ShapeDtypeStruct((2, 16), jnp.float32),
        scratch_types=[
            pltpu.VMEM((_B,), jnp.int32),
            pltpu.VMEM((_C,), jnp.float32),
            pltpu.VMEM((_C,), jnp.float32),
            pltpu.VMEM((16,), jnp.float32),
            pltpu.VMEM((16,), jnp.int32),
            pltpu.VMEM((16,), jnp.float32),
            pltpu.VMEM((16,), jnp.float32),
            pltpu.VMEM((16,), jnp.float32),
            pltpu.VMEM_SHARED((16, 16), jnp.float32),
            pltpu.SemaphoreType.DMA,
            pltpu.SemaphoreType.DMA,
        ],
        compiler_params=pltpu.CompilerParams(use_tc_tiling_on_sc=False),
    )(_body)
    return f(outputs, targets)


def kernel(outputs, targets):
    out = _run(outputs, targets)
    s = out[0] + out[1]
    return (s[0], s[1])


# speedup vs baseline: 2.1721x; 1.2586x over previous
import functools

import jax
import jax.numpy as jnp
from jax import lax
from jax.experimental import pallas as pl
from jax.experimental.pallas import tpu as pltpu
from jax.experimental.pallas import tpu_sc as plsc

_B = 128
_V = 100000
_RB = 8
_HALF_T = 391
_HALF_C = _HALF_T * 128
_C = 2048
_NFULL = 24
_TAIL_C = _HALF_C - _NFULL * _C
_GV = 16
_NG = (_C // 16) // _GV
_NTG = (_TAIL_C // 16) // 8


def _body(x_hbm, tgt_hbm, out_ref, tgt_v, bufa_v, bufb_v, buft_v, tvblk_v,
          acc8_v, prt_v, tmp16_v, part_v, red_v, tot_v, shared_i, shared_f,
          sem0, sem1):
    cid = lax.axis_index("c")
    sid = lax.axis_index("s")
    wid = cid * 16 + sid
    rb = wid // 2
    h = wid % 2
    rbase = rb * _RB
    hs = h * _HALF_C

    pltpu.sync_copy(tgt_hbm, tgt_v)
    iota = lax.iota(jnp.int32, 16)
    zero16i = jnp.zeros((16,), jnp.int32)
    one16i = jnp.ones((16,), jnp.int32)

    ts = []
    tvs = []
    for i in range(_RB):
        ts.append(tgt_v[pl.ds(rbase + i, 1)][0])
    for i in range(_RB):
        tcol = (ts[i] // 128) * 128
        pltpu.sync_copy(
            x_hbm.at[pl.ds(pl.multiple_of(rbase, 8), _RB),
                     pl.ds(pl.multiple_of(tcol, 128), 128)],
            tvblk_v)
        tvs.append(tvblk_v[i, pl.ds(ts[i] - tcol, 1)][0])

    for i in range(_RB):
        acc8_v[pl.ds(i * 16, 16)] = zero16i

    def start(c, buf, sem):
        cs = pl.multiple_of(hs + c * _C, 128)
        pltpu.async_copy(
            x_hbm.at[pl.ds(pl.multiple_of(rbase, 8), _RB), pl.ds(cs, _C)],
            buf, sem)

    def wait(buf, sem):
        pltpu.make_async_copy(
            x_hbm.at[pl.ds(0, _RB), pl.ds(0, _C)], buf, sem).wait()

    def count4(load, pred, n):
        a = [zero16i, zero16i, zero16i, zero16i]
        for k in range(n):
            a[k % 4] = a[k % 4] + jnp.where(pred(load(k)), one16i, zero16i)
        return (a[0] + a[1]) + (a[2] + a[3])

    def process(c, buf):
        s = hs + c * _C
        e = s + _C
        for i in range(_RB):
            t_i = ts[i]
            tv_i = tvs[i]

            def grp_gt(g, acc):
                gb = g * (_GV * 16)
                return acc + count4(
                    lambda k: buf[i, pl.ds(gb + k * 16, 16)],
                    lambda v: v > tv_i, _GV)

            acc = lax.fori_loop(0, _NG, grp_gt, zero16i)
            acc8_v[pl.ds(i * 16, 16)] = acc8_v[pl.ds(i * 16, 16)] + acc

            @pl.when(e <= t_i)
            def _():
                def grp_eq(g, acc):
                    gb = g * (_GV * 16)
                    return acc + count4(
                        lambda k: buf[i, pl.ds(gb + k * 16, 16)],
                        lambda v: v == tv_i, _GV)

                acc = lax.fori_loop(0, _NG, grp_eq, zero16i)
                acc8_v[pl.ds(i * 16, 16)] = acc8_v[pl.ds(i * 16, 16)] + acc

            @pl.when((s < t_i) & (t_i < e))
            def _():
                def grp_mid(g, acc):
                    gb = g * (_GV * 16)
                    col = iota + (s + gb)
                    a = zero16i
                    for k in range(_GV):
                        v = buf[i, pl.ds(gb + k * 16, 16)]
                        m = (v == tv_i) & (col < t_i)
                        a = a + jnp.where(m, one16i, zero16i)
                        col = col + 16
                    return acc + a

                acc = lax.fori_loop(0, _NG, grp_mid, zero16i)
                acc8_v[pl.ds(i * 16, 16)] = acc8_v[pl.ds(i * 16, 16)] + acc

    start(0, bufa_v, sem0)
    start(1, bufb_v, sem1)

    def pair(p, _):
        c0 = 2 * p
        wait(bufa_v, sem0)
        process(c0, bufa_v)

        @pl.when(c0 + 2 < _NFULL)
        def _():
            start(c0 + 2, bufa_v, sem0)

        wait(bufb_v, sem1)
        process(c0 + 1, bufb_v)

        @pl.when(c0 + 3 < _NFULL)
        def _():
            start(c0 + 3, bufb_v, sem1)

        return 0

    lax.fori_loop(0, _NFULL // 2, pair, 0)

    tts = hs + _NFULL * _C
    pltpu.sync_copy(
        x_hbm.at[pl.ds(pl.multiple_of(rbase, 8), _RB),
                 pl.ds(pl.multiple_of(tts, 128), _TAIL_C)],
        buft_v)
    for i in range(_RB):
        t_i = ts[i]
        tv_i = tvs[i]

        def tgrp(g, acc):
            gb = g * 128
            col = iota + (tts + gb)
            a = zero16i
            for k in range(8):
                v = buft_v[i, pl.ds(gb + k * 16, 16)]
                m = ((v > tv_i) & (col < _V)) | ((v == tv_i) & (col < t_i))
                a = a + jnp.where(m, one16i, zero16i)
                col = col + 16
            return acc + a

        acc = lax.fori_loop(0, _NTG, tgrp, zero16i)
        acc8_v[pl.ds(i * 16, 16)] = acc8_v[pl.ds(i * 16, 16)] + acc

    pltpu.sync_copy(acc8_v, shared_i.at[sid])
    plsc.subcore_barrier()

    part_v[...] = jnp.zeros((16,), jnp.float32)

    @pl.when(sid % 2 == 0)
    def _():
        pltpu.sync_copy(shared_i.at[sid + 1], prt_v)
        top1 = jnp.float32(0.0)
        top5 = jnp.float32(0.0)
        for i in range(_RB):
            tmp16_v[...] = (acc8_v[pl.ds(i * 16, 16)]
                            + prt_v[pl.ds(i * 16, 16)])
            rank = tmp16_v[pl.ds(0, 1)][0]
            for q in range(1, 16):
                rank = rank + tmp16_v[pl.ds(q, 1)][0]
            top1 = top1 + jnp.where(rank < 1, 1.0, 0.0).astype(jnp.float32)
            top5 = top5 + jnp.where(rank < 5, 1.0, 0.0).astype(jnp.float32)
        part_v[...] = jnp.where(iota == 0, top1,
                                jnp.where(iota == 1, top5,
                                          jnp.zeros((16,), jnp.float32)))

    pltpu.sync_copy(part_v, shared_f.at[sid])
    plsc.subcore_barrier()

    @pl.when(sid == 0)
    def _():
        def red(i, a):
            pltpu.sync_copy(shared_f.at[i], red_v)
            return a + red_v[...]

        tot = lax.fori_loop(0, 16, red, jnp.zeros((16,), jnp.float32))
        tot_v[...] = tot
        pltpu.sync_copy(tot_v, out_ref.at[cid])


@jax.jit
def _run(outputs, targets):
    mesh = plsc.VectorSubcoreMesh(core_axis_name="c", subcore_axis_name="s")
    f = functools.partial(
        pl.kernel,
        mesh=mesh,
        out_type=jax.ShapeDtypeStruct((2, 16), jnp.float32),
        scratch_types=[
            pltpu.VMEM((_B,), jnp.int32),
            pltpu.VMEM((_RB, _C), jnp.float32),
            pltpu.VMEM((_RB, _C), jnp.float32),
            pltpu.VMEM((_RB, _TAIL_C), jnp.float32),
            pltpu.VMEM((_RB, 128), jnp.float32),
            pltpu.VMEM((128,), jnp.int32),
            pltpu.VMEM((128,), jnp.int32),
            pltpu.VMEM((16,), jnp.int32),
            pltpu.VMEM((16,), jnp.float32),
            pltpu.VMEM((16,), jnp.float32),
            pltpu.VMEM((16,), jnp.float32),
            pltpu.VMEM_SHARED((16, 128), jnp.int32),
            pltpu.VMEM_SHARED((16, 16), jnp.float32),
            pltpu.SemaphoreType.DMA,
            pltpu.SemaphoreType.DMA,
        ],
    )(_body)
    return f(outputs, targets)


def kernel(outputs, targets):
    out = _run(outputs, targets)
    s = out[0] + out[1]
    return (s[0], s[1])
